# baseline (speedup 1.0000x reference)
import jax
import jax.numpy as jnp
from jax import lax
from jax.experimental import pallas as pl
from jax.experimental.pallas import tpu as pltpu
from jax.experimental.pallas import tpu_sc as plsc

N = 10000
D_IN = 128
D_H = 256
D_OUT = 128
E = 320000
EL = 20000

NC = 2
NS = 16
W = 128
NB = 2
SG = 16

EP = 327680
NWIN = EP // W
NP = 10240
PAD_ROW = N
RPS = NP // NS
NWIN_HALF = NWIN // NS
NWIN_FULL = NWIN // (NC * NS)
ELP = 40960
NWIN_DEC = ELP // W // (NC * NS)

ROW_BLK = 2000

_MESH = plsc.VectorSubcoreMesh(core_axis_name="c", subcore_axis_name="s")
_f32 = jnp.float32

_AGG_SCRATCH = (
    [pltpu.VMEM_SHARED((NP, 128), _f32)]
    + [pltpu.VMEM((W, 128), _f32)]
    + [pltpu.VMEM((SG, W), jnp.int32) for _ in range(2)]
    + [pltpu.SemaphoreType.DMA]
)

_DEG_SCRATCH = (
    [pltpu.VMEM_SHARED((NP, 128), _f32)]
    + [pltpu.VMEM((W, 128), _f32) for _ in range(2)]
    + [pltpu.VMEM((SG, W), jnp.int32)]
)


def _zero_acc(z_hbm, acc, buf, sid):
    @pl.loop(0, RPS // W)
    def _(k):
        r = sid * RPS + k * W
        pltpu.sync_copy(z_hbm.at[pl.ds(r, W)], buf)
        pltpu.sync_copy(buf, acc.at[pl.ds(r, W)])


def _dump_acc(acc, out_hbm, buf, sid):
    @pl.loop(0, RPS // W)
    def _(k):
        r = sid * RPS + k * W
        pltpu.sync_copy(acc.at[pl.ds(r, W)], buf)
        pltpu.sync_copy(buf, out_hbm.at[pl.ds(r, W)])


def _agg_run(tab_hbm, out_hbm, srcw_hbm, dstw_hbm, z_hbm, acc, rows, isrc,
             idst, sem, sid, gbase, nwin):
    _zero_acc(z_hbm, acc, rows, sid)
    plsc.subcore_barrier()

    @pl.loop(0, nwin // SG)
    def _(s):
        g0 = gbase + s * SG
        pltpu.sync_copy(srcw_hbm.at[pl.ds(g0, SG)], isrc)
        pltpu.sync_copy(dstw_hbm.at[pl.ds(g0, SG)], idst)

        @pl.loop(0, SG)
        def _(k):
            pltpu.async_copy(tab_hbm.at[isrc.at[k]], rows, sem).wait()
            pltpu.sync_copy(rows, acc.at[idst.at[k]], add=True)

    plsc.subcore_barrier()
    _dump_acc(acc, out_hbm, rows, sid)


def _deg_body(dstw_hbm, z_hbm, o_hbm, out_hbm, acc, r0, r1, idst):
    c = lax.axis_index("c")
    sid = lax.axis_index("s")
    wid = sid * NC + c
    ones = r0
    buf = r1

    pltpu.sync_copy(o_hbm, ones)
    _zero_acc(z_hbm, acc, buf, sid)
    plsc.subcore_barrier()

    gbase = wid * NWIN_FULL

    @pl.loop(0, NWIN_FULL // SG)
    def _(s):
        pltpu.sync_copy(dstw_hbm.at[pl.ds(gbase + s * SG, SG)], idst)

        @pl.loop(0, SG)
        def _(k):
            pltpu.sync_copy(ones, acc.at[idst.at[k]], add=True)

    plsc.subcore_barrier()

    @pl.loop(0, RPS // W)
    def _(k):
        r = sid * RPS + k * W
        pltpu.sync_copy(acc.at[pl.ds(r, W)], buf)
        pltpu.sync_copy(buf, out_hbm.at[c, pl.ds(r, W)])


_deg_call = pl.kernel(
    _deg_body,
    out_type=jax.ShapeDtypeStruct((NC, NP, 128), _f32),
    mesh=_MESH,
    scratch_types=_DEG_SCRATCH,
)


def _agg1_body(tab_a, tab_b, srcw_hbm, dstw_hbm, z_hbm, out_a, out_b, acc,
               rows, isrc, idst, sem):
    c = lax.axis_index("c")
    sid = lax.axis_index("s")
    gbase = sid * NWIN_HALF

    @pl.when(c == 0)
    def _():
        _agg_run(tab_a, out_a, srcw_hbm, dstw_hbm, z_hbm, acc, rows, isrc,
                 idst, sem, sid, gbase, NWIN_HALF)

    @pl.when(c == 1)
    def _():
        _agg_run(tab_b, out_b, srcw_hbm, dstw_hbm, z_hbm, acc, rows, isrc,
                 idst, sem, sid, gbase, NWIN_HALF)


_agg1_call = pl.kernel(
    _agg1_body,
    out_type=[
        jax.ShapeDtypeStruct((NP, 128), _f32),
        jax.ShapeDtypeStruct((NP, 128), _f32),
    ],
    mesh=_MESH,
    scratch_types=_AGG_SCRATCH,
)


def _agg2_body(tab_hbm, srcw_hbm, dstw_hbm, z_hbm, out_hbm, acc, rows, isrc,
               idst, sem):
    c = lax.axis_index("c")
    sid = lax.axis_index("s")
    wid = sid * NC + c

    _agg_run(tab_hbm, out_hbm.at[c], srcw_hbm, dstw_hbm, z_hbm, acc, rows,
             isrc, idst, sem, sid, wid * NWIN_FULL, NWIN_FULL)


_agg2_call = pl.kernel(
    _agg2_body,
    out_type=jax.ShapeDtypeStruct((NC, NP, 128), _f32),
    mesh=_MESH,
    scratch_types=_AGG_SCRATCH,
)


def _dec_body(tab_hbm, idx_hbm, out_hbm, rows, gidx, sem):
    c = lax.axis_index("c")
    sid = lax.axis_index("s")
    wid = sid * NC + c
    gbase = wid * NWIN_DEC

    @pl.loop(0, NWIN_DEC)
    def _(j):
        win = gbase + j
        pltpu.sync_copy(idx_hbm.at[win], gidx.at[0])
        pltpu.async_copy(tab_hbm.at[gidx.at[0]], rows, sem).wait()
        pltpu.sync_copy(rows, out_hbm.at[pl.ds(win * W, W)])


_dec_call = pl.kernel(
    _dec_body,
    out_type=jax.ShapeDtypeStruct((ELP, 128), _f32),
    mesh=_MESH,
    scratch_types=[
        pltpu.VMEM((W, 128), _f32),
        pltpu.VMEM((1, W), jnp.int32),
        pltpu.SemaphoreType.DMA,
    ],
)


def _mms_body(x_ref, w_ref, dega_ref, degb_ref, hs_ref, dinv_ref):
    dinv = jax.lax.rsqrt(dega_ref[...] + degb_ref[...])
    dinv_ref[...] = dinv
    h = jnp.dot(x_ref[...], w_ref[...], preferred_element_type=jnp.float32)
    hs_ref[...] = h * dinv


def _tc_mm_scale(x, w, dega, degb):
    n, k = x.shape
    m = w.shape[1]
    return pl.pallas_call(
        _mms_body,
        grid=(n // ROW_BLK,),
        in_specs=[
            pl.BlockSpec((ROW_BLK, k), lambda i: (i, 0)),
            pl.BlockSpec((k, m), lambda i: (0, 0)),
            pl.BlockSpec((ROW_BLK, 1), lambda i: (i, 0)),
            pl.BlockSpec((ROW_BLK, 1), lambda i: (i, 0)),
        ],
        out_specs=[
            pl.BlockSpec((ROW_BLK, m), lambda i: (i, 0)),
            pl.BlockSpec((ROW_BLK, 1), lambda i: (i, 0)),
        ],
        out_shape=[
            jax.ShapeDtypeStruct((n, m), jnp.float32),
            jax.ShapeDtypeStruct((n, 1), jnp.float32),
        ],
    )(x, w, dega, degb)


def _mid_body(agga_ref, aggb_ref, hs1_ref, dinv_ref, b1_ref, w2_ref, hs2_ref):
    agg = jnp.concatenate([agga_ref[...], aggb_ref[...]], axis=-1)
    out1 = jax.nn.relu(dinv_ref[...] * (agg + hs1_ref[...]) + b1_ref[...])
    h2 = jnp.dot(out1, w2_ref[...], preferred_element_type=jnp.float32)
    hs2_ref[...] = h2 * dinv_ref[...]


def _tc_mid(agga, aggb, hs1, dinv, b1, W2):
    n = agga.shape[0]
    return pl.pallas_call(
        _mid_body,
        grid=(n // ROW_BLK,),
        in_specs=[
            pl.BlockSpec((ROW_BLK, 128), lambda i: (i, 0)),
            pl.BlockSpec((ROW_BLK, 128), lambda i: (i, 0)),
            pl.BlockSpec((ROW_BLK, D_H), lambda i: (i, 0)),
            pl.BlockSpec((ROW_BLK, 1), lambda i: (i, 0)),
            pl.BlockSpec((1, D_H), lambda i: (0, 0)),
            pl.BlockSpec((D_H, D_OUT), lambda i: (0, 0)),
        ],
        out_specs=pl.BlockSpec((ROW_BLK, D_OUT), lambda i: (i, 0)),
        out_shape=jax.ShapeDtypeStruct((n, D_OUT), jnp.float32),
    )(agga, aggb, hs1, dinv, b1, W2)


def _z_body(p0_ref, p1_ref, hs2_ref, dinv_ref, b2_ref, z_ref):
    z_ref[...] = (dinv_ref[...] * (p0_ref[...] + p1_ref[...] + hs2_ref[...])
                  + b2_ref[...])


def _tc_z(p0, p1, hs2, dinv, b2):
    n = p0.shape[0]
    return pl.pallas_call(
        _z_body,
        grid=(n // ROW_BLK,),
        in_specs=[
            pl.BlockSpec((ROW_BLK, D_OUT), lambda i: (i, 0)),
            pl.BlockSpec((ROW_BLK, D_OUT), lambda i: (i, 0)),
            pl.BlockSpec((ROW_BLK, D_OUT), lambda i: (i, 0)),
            pl.BlockSpec((ROW_BLK, 1), lambda i: (i, 0)),
            pl.BlockSpec((1, D_OUT), lambda i: (0, 0)),
        ],
        out_specs=pl.BlockSpec((ROW_BLK, D_OUT), lambda i: (i, 0)),
        out_shape=jax.ShapeDtypeStruct((n, D_OUT), jnp.float32),
    )(p0, p1, hs2, dinv, b2)


def _dot_body(zs_ref, zd_ref, o_ref):
    o_ref[...] = jnp.sum(zs_ref[...] * zd_ref[...], axis=-1, keepdims=True)


def _tc_dot(zs, zd):
    n = zs.shape[0]
    return pl.pallas_call(
        _dot_body,
        grid=(n // ROW_BLK,),
        in_specs=[
            pl.BlockSpec((ROW_BLK, D_OUT), lambda i: (i, 0)),
            pl.BlockSpec((ROW_BLK, D_OUT), lambda i: (i, 0)),
        ],
        out_specs=pl.BlockSpec((ROW_BLK, 1), lambda i: (i, 0)),
        out_shape=jax.ShapeDtypeStruct((n, 1), jnp.float32),
    )(zs, zd)


def _pad_rows(a):
    return jnp.concatenate(
        [a, jnp.zeros((NP - a.shape[0], a.shape[1]), a.dtype)], axis=0)


def kernel(x, edge_index, edge_label_index, W1, b1, W2, b2):
    epad = jnp.full((1, EP - E), PAD_ROW, jnp.int32)
    src_w = jnp.concatenate([edge_index[:1], epad], axis=1).reshape(NWIN, W)
    dst_w = jnp.concatenate([edge_index[1:2], epad], axis=1).reshape(NWIN, W)
    z128 = jnp.zeros((NP, 128), jnp.float32)
    ones = jnp.ones((W, 128), jnp.float32)

    degacc = _deg_call(dst_w, z128, ones)
    dega = degacc[0, :N, :1] + 1.0
    degb = degacc[1, :N, :1]

    hs1, dinv = _tc_mm_scale(x, W1, dega, degb)

    hs1p = _pad_rows(hs1[:, :128])
    hs1q = _pad_rows(hs1[:, 128:])
    agg_a, agg_b = _agg1_call(hs1p, hs1q, src_w, dst_w, z128)

    hs2 = _tc_mid(agg_a[:N], agg_b[:N], hs1, dinv, b1[None, :], W2)

    hs2p = _pad_rows(hs2)
    agg2 = _agg2_call(hs2p, src_w, dst_w, z128)

    z = _tc_z(agg2[0, :N], agg2[1, :N], hs2, dinv, b2[None, :])

    zp = _pad_rows(z)
    lpad = jnp.full((ELP - 2 * EL,), PAD_ROW, jnp.int32)
    dec_idx = jnp.concatenate(
        [edge_label_index[0], edge_label_index[1], lpad]).reshape(ELP // W, W)
    rows = _dec_call(zp, dec_idx)

    return _tc_dot(rows[:EL], rows[EL:2 * EL])[:, 0]
